# baseline (speedup 1.0000x reference)
import functools

import jax
import jax.numpy as jnp
import numpy as np
from jax import lax
from jax.experimental import pallas as pl
from jax.experimental.pallas import tpu as pltpu
from jax.experimental.pallas import tpu_sc as plsc

_SQ = float(np.sqrt(np.float32(1.0 + 1e-5)))
_K1 = 20
_NPOOL = 256


def _act(h):
    h = h / _SQ
    return jnp.where(h >= 0, h, 0.2 * h)


def _knn_neg(xc):
    xt = xc.transpose(0, 2, 1)
    inner = jnp.matmul(xt, xc)
    xx = jnp.sum(xc ** 2, axis=1)
    return 2.0 * inner - xx[:, :, None] - xx[:, None, :]


def _gfeat(xc, idx):
    x_t = xc.transpose(0, 2, 1)
    feat = jax.vmap(lambda xt_, ii: xt_[ii])(x_t, idx)
    center = x_t[:, :, None, :]
    out = jnp.concatenate(
        [feat - center, jnp.broadcast_to(center, feat.shape)], axis=-1
    )
    return out.transpose(0, 3, 1, 2)


def _blk2(W, h):
    h = jnp.einsum("oc,bcnk->bonk", W, h)
    return jax.nn.leaky_relu(h / jnp.sqrt(1.0 + 1e-5), negative_slope=0.2)


def _topk_kern(d_ref, oi_ref, *, kk):
    cur = d_ref[0]
    n = cur.shape[1]
    iota = lax.broadcasted_iota(jnp.int32, cur.shape, 1)
    cols = []
    for _ in range(kk):
        m = jnp.max(cur, axis=1, keepdims=True)
        am = jnp.min(jnp.where(cur == m, iota, n), axis=1)
        cols.append(am)
        cur = jnp.where(iota == am[:, None], -jnp.inf, cur)
    oi_ref[0] = jnp.stack(cols, axis=1)


def _topk_idx(negd, kk, tile):
    B, M, N = negd.shape
    return pl.pallas_call(
        functools.partial(_topk_kern, kk=kk),
        grid=(B, M // tile),
        in_specs=[pl.BlockSpec((1, tile, N), lambda b, i: (b, i, 0))],
        out_specs=pl.BlockSpec((1, tile, kk), lambda b, i: (b, i, 0)),
        out_shape=jax.ShapeDtypeStruct((B, M, kk), jnp.int32),
    )(negd)


def _dist_topk_kern(q_ref, k_ref, oi_ref, *, kk):
    q = q_ref[0]
    keys = k_ref[0]
    n = keys.shape[0]
    inner = lax.dot_general(
        q, keys, (((1,), (1,)), ((), ())), preferred_element_type=jnp.float32
    )
    cur = 2.0 * inner - jnp.sum(q * q, 1, keepdims=True) - jnp.sum(keys * keys, 1)[None, :]
    iota = lax.broadcasted_iota(jnp.int32, cur.shape, 1)
    cols = []
    for _ in range(kk):
        m = jnp.max(cur, axis=1, keepdims=True)
        am = jnp.min(jnp.where(cur == m, iota, n), axis=1)
        cols.append(am)
        cur = jnp.where(iota == am[:, None], -jnp.inf, cur)
    oi_ref[0] = jnp.stack(cols, axis=1)


def _dist_topk(q, k, kk, tile):
    B, M, C = q.shape
    N = k.shape[1]
    return pl.pallas_call(
        functools.partial(_dist_topk_kern, kk=kk),
        grid=(B, M // tile),
        in_specs=[
            pl.BlockSpec((1, tile, C), lambda b, i: (b, i, 0)),
            pl.BlockSpec((1, N, C), lambda b, i: (b, 0, 0)),
        ],
        out_specs=pl.BlockSpec((1, tile, kk), lambda b, i: (b, i, 0)),
        out_shape=jax.ShapeDtypeStruct((B, M, kk), jnp.int32),
    )(q, k)


def _ec_kern(g_ref, c_ref, w1_ref, w2_ref, o_ref, *, k, mode):
    acc = None
    for j in range(k):
        gj = g_ref[0, j]
        if mode == "raw":
            h = gj
        else:
            c = c_ref[0]
            e = jnp.concatenate([gj - c, c], axis=1)
            h = _act(jnp.dot(e, w1_ref[...], preferred_element_type=jnp.float32))
            if mode == "conv2":
                h = _act(jnp.dot(h, w2_ref[...], preferred_element_type=jnp.float32))
        acc = h if acc is None else jnp.maximum(acc, h)
    o_ref[0] = acc


def _edgeconv(g, center, w1_t, w2_t, mode, tile):
    B, k, N, C = g.shape
    if center is None:
        center = jnp.zeros((B, N, C), jnp.float32)
    if w1_t is None:
        w1_t = jnp.zeros((2 * C, 64), jnp.float32)
    if w2_t is None:
        w2_t = jnp.zeros((64, 64), jnp.float32)
    Co = C if mode == "raw" else w1_t.shape[1]
    return pl.pallas_call(
        functools.partial(_ec_kern, k=k, mode=mode),
        grid=(B, N // tile),
        in_specs=[
            pl.BlockSpec((1, k, tile, C), lambda b, i: (b, 0, i, 0)),
            pl.BlockSpec((1, tile, C), lambda b, i: (b, i, 0)),
            pl.BlockSpec(w1_t.shape, lambda b, i: (0, 0)),
            pl.BlockSpec(w2_t.shape, lambda b, i: (0, 0)),
        ],
        out_specs=pl.BlockSpec((1, tile, Co), lambda b, i: (b, i, 0)),
        out_shape=jax.ShapeDtypeStruct((B, N, Co), jnp.float32),
    )(g, center, w1_t, w2_t)


def _mm_kern(x_ref, w_ref, o_ref, *, act):
    acc = jnp.dot(x_ref[0], w_ref[...], preferred_element_type=jnp.float32)
    o_ref[0] = _act(acc) if act else acc


def _mm(x, w_t, act, tile):
    B, N, Ci = x.shape
    Co = w_t.shape[1]
    return pl.pallas_call(
        functools.partial(_mm_kern, act=act),
        grid=(B, N // tile),
        in_specs=[
            pl.BlockSpec((1, tile, Ci), lambda b, i: (b, i, 0)),
            pl.BlockSpec((Ci, Co), lambda b, i: (0, 0)),
        ],
        out_specs=pl.BlockSpec((1, tile, Co), lambda b, i: (b, i, 0)),
        out_shape=jax.ShapeDtypeStruct((B, N, Co), jnp.float32),
    )(x, w_t)


def _mm2_kern(x_ref, y_ref, wa_ref, wb_ref, o_ref, *, act):
    acc = jnp.dot(x_ref[0], wa_ref[...], preferred_element_type=jnp.float32)
    acc = acc + jnp.dot(y_ref[0], wb_ref[...], preferred_element_type=jnp.float32)
    o_ref[0] = _act(acc) if act else acc


def _mm2(x, y, wa_t, wb_t, act, tile):
    B, N, Ca = x.shape
    Cb = y.shape[2]
    Co = wa_t.shape[1]
    return pl.pallas_call(
        functools.partial(_mm2_kern, act=act),
        grid=(B, N // tile),
        in_specs=[
            pl.BlockSpec((1, tile, Ca), lambda b, i: (b, i, 0)),
            pl.BlockSpec((1, tile, Cb), lambda b, i: (b, i, 0)),
            pl.BlockSpec((Ca, Co), lambda b, i: (0, 0)),
            pl.BlockSpec((Cb, Co), lambda b, i: (0, 0)),
        ],
        out_specs=pl.BlockSpec((1, tile, Co), lambda b, i: (b, i, 0)),
        out_shape=jax.ShapeDtypeStruct((B, N, Co), jnp.float32),
    )(x, y, wa_t, wb_t)


def _top3_kern(q_ref, n_ref, oi_ref, ow_ref):
    q = q_ref[0]
    nodes = n_ref[0]
    M = nodes.shape[0]
    inner = lax.dot_general(
        q, nodes, (((1,), (1,)), ((), ())), preferred_element_type=jnp.float32
    )
    neg = 2.0 * inner - jnp.sum(q * q, 1, keepdims=True) - jnp.sum(nodes * nodes, 1)[None, :]
    iota = lax.broadcasted_iota(jnp.int32, neg.shape, 1)
    vals = []
    cur = neg
    for t in range(3):
        m = jnp.max(cur, axis=1, keepdims=True)
        amax = jnp.min(jnp.where(cur == m, iota, M), axis=1)
        oi_ref[0, t] = amax
        vals.append(m[:, 0])
        cur = jnp.where(iota == amax[:, None], -jnp.inf, cur)
    w = [1.0 / (jnp.maximum(-v, 0.0) + 1e-8) for v in vals]
    tot = w[0] + w[1] + w[2]
    for t in range(3):
        ow_ref[0, t] = w[t] / tot


def _unpool_top3(q, nodes, tile):
    B, N, C = q.shape
    M = nodes.shape[1]
    return pl.pallas_call(
        _top3_kern,
        grid=(B, N // tile),
        in_specs=[
            pl.BlockSpec((1, tile, C), lambda b, i: (b, i, 0)),
            pl.BlockSpec((1, M, C), lambda b, i: (b, 0, 0)),
        ],
        out_specs=[
            pl.BlockSpec((1, 3, tile), lambda b, i: (b, 0, i)),
            pl.BlockSpec((1, 3, tile), lambda b, i: (b, 0, i)),
        ],
        out_shape=[
            jax.ShapeDtypeStruct((B, 3, N), jnp.int32),
            jax.ShapeDtypeStruct((B, 3, N), jnp.float32),
        ],
    )(q, nodes)


def _final_kern(g_ref, wu_ref, x2_ref, x1_ref, wa_ref, wb_ref, wc_ref, wd_ref, o_ref):
    wu = wu_ref[0]
    hat = (
        wu[:, 0:1] * g_ref[0, 0]
        + wu[:, 1:2] * g_ref[0, 1]
        + wu[:, 2:3] * g_ref[0, 2]
    )
    h10 = jnp.dot(hat, wa_ref[...], preferred_element_type=jnp.float32)
    h10 = h10 + jnp.dot(x2_ref[0], wb_ref[...], preferred_element_type=jnp.float32)
    h10 = _act(h10)
    out = jnp.dot(h10, wc_ref[...], preferred_element_type=jnp.float32)
    out = out + jnp.dot(x1_ref[0], wd_ref[...], preferred_element_type=jnp.float32)
    o_ref[0] = out


def _final(g, wu, x2, x1, wa_t, wb_t, wc_t, wd_t, tile):
    B, _, N, D = g.shape
    Co = wc_t.shape[1]
    return pl.pallas_call(
        _final_kern,
        grid=(B, N // tile),
        in_specs=[
            pl.BlockSpec((1, 3, tile, D), lambda b, i: (b, 0, i, 0)),
            pl.BlockSpec((1, tile, 3), lambda b, i: (b, i, 0)),
            pl.BlockSpec((1, tile, 64), lambda b, i: (b, i, 0)),
            pl.BlockSpec((1, tile, 64), lambda b, i: (b, i, 0)),
            pl.BlockSpec((D, 128), lambda b, i: (0, 0)),
            pl.BlockSpec((64, 128), lambda b, i: (0, 0)),
            pl.BlockSpec((128, Co), lambda b, i: (0, 0)),
            pl.BlockSpec((64, Co), lambda b, i: (0, 0)),
        ],
        out_specs=pl.BlockSpec((1, tile, Co), lambda b, i: (b, i, 0)),
        out_shape=jax.ShapeDtypeStruct((B, N, Co), jnp.float32),
    )(g, wu, x2, x1, wa_t, wb_t, wc_t, wd_t)


def _gather_rows(table, idx, B, k, N):
    D = table.shape[1]
    flat = idx.reshape(-1)
    out = jnp.take(table, flat, axis=0)
    return out.reshape(B, k, N, D)


def kernel(x, W1, W2, W3, W4, W5, W6, W8, W9, W10, W11, p):
    B, _, N = x.shape
    k1 = _K1

    x_tpA = jnp.pad(x.transpose(0, 2, 1), ((0, 0), (0, 0), (0, 7)))
    idx1 = _dist_topk(x_tpA, x_tpA, k1, tile=256)
    h = _blk2(W2, _blk2(W1, _gfeat(x, idx1)))
    x1 = jnp.max(h, axis=-1)

    idx2 = _dist_topk(x1.transpose(0, 2, 1), x1.transpose(0, 2, 1), k1, tile=256)
    h = _blk2(W4, _blk2(W3, _gfeat(x1, idx2)))
    x2 = jnp.max(h, axis=-1)

    xyz = x[:, :3, :]
    scores = jnp.einsum("c,bcn->bn", p, x2) / (jnp.linalg.norm(p) + 1e-8)
    values, idxp = lax.top_k(scores, _NPOOL)
    feat = jnp.take_along_axis(x2, idxp[:, None, :], axis=2)
    node_feature = feat * jnp.tanh(values)[:, None, :]
    n2 = jnp.take_along_axis(xyz, idxp[:, None, :], axis=2)

    x_t = x.transpose(0, 2, 1)
    xyz_p = jnp.pad(x_t[:, :, :3], ((0, 0), (0, 0), (0, 5)))
    n2_t = n2.transpose(0, 2, 1)
    n2_p = jnp.pad(n2_t, ((0, 0), (0, 0), (0, 5)))
    x1t = x1.transpose(0, 2, 1)
    x2t = x2.transpose(0, 2, 1)
    boff1 = (jnp.arange(B, dtype=jnp.int32) * N)[:, None, None]
    boff3 = (jnp.arange(B, dtype=jnp.int32) * _NPOOL)[:, None, None]

    idxa = _dist_topk(n2_p, xyz_p, k1 // 2, tile=256)
    ga = _gather_rows(
        x2t.reshape(B * N, 64), idxa.transpose(0, 2, 1) + boff1, B, k1 // 2, _NPOOL
    )
    agg = _edgeconv(ga, None, None, None, mode="raw", tile=256)

    nf2 = jnp.concatenate([node_feature.transpose(0, 2, 1), agg], axis=2)

    idx3 = _dist_topk(nf2, nf2, k1 // 2, tile=256)
    g3 = _gather_rows(
        nf2.reshape(B * _NPOOL, 128), idx3.transpose(0, 2, 1) + boff3, B, k1 // 2, _NPOOL
    )
    x3 = _edgeconv(g3, nf2, W5.T, None, mode="act", tile=256)

    x4 = _mm(x3, W6.T, act=True, tile=256)
    h8 = _mm(x4, W8.T, act=True, tile=256)
    h9 = _mm2(h8, x3, W9[:, :256].T, W9[:, 256:].T, act=True, tile=256)

    idxu, wu = _unpool_top3(xyz_p, n2_p, tile=512)
    gu = _gather_rows(h9.reshape(B * _NPOOL, 256), idxu + boff3, B, 3, N)
    out_t = _final(
        gu, wu.transpose(0, 2, 1), x2t, x1t,
        W10[:, :256].T, W10[:, 256:].T, W11[:, :128].T, W11[:, 128:].T,
        tile=512,
    )
    out = out_t.transpose(0, 2, 1)

    return (out, scores, n2, n2)


# speedup vs baseline: 3.6133x; 3.6133x over previous
import functools

import jax
import jax.numpy as jnp
import numpy as np
from jax import lax
from jax.experimental import pallas as pl
from jax.experimental.pallas import tpu as pltpu
from jax.experimental.pallas import tpu_sc as plsc

_SQ = float(np.sqrt(np.float32(1.0 + 1e-5)))
_K1 = 20
_NPOOL = 256


def _act(h):
    h = h / _SQ
    return jnp.where(h >= 0, h, 0.2 * h)


def _knn_neg(xc):
    xt = xc.transpose(0, 2, 1)
    inner = jnp.matmul(xt, xc)
    xx = jnp.sum(xc ** 2, axis=1)
    return 2.0 * inner - xx[:, :, None] - xx[:, None, :]


def _gfeat(xc, idx):
    B, C, N = xc.shape
    k = idx.shape[2]
    x_t = xc.transpose(0, 2, 1)
    boff = (jnp.arange(B, dtype=jnp.int32) * N)[:, None, None]
    flat = (idx + boff).reshape(-1)
    feat = _sc_gather(x_t.reshape(B * N, C), flat).reshape(B, N, k, C)
    center = x_t[:, :, None, :]
    out = jnp.concatenate(
        [feat - center, jnp.broadcast_to(center, feat.shape)], axis=-1
    )
    return out.transpose(0, 3, 1, 2)


def _blk2(W, h):
    h = jnp.einsum("oc,bcnk->bonk", W, h)
    return jax.nn.leaky_relu(h / jnp.sqrt(1.0 + 1e-5), negative_slope=0.2)


def _topk_kern(d_ref, oi_ref, *, kk):
    cur = d_ref[0]
    n = cur.shape[1]
    iota = lax.broadcasted_iota(jnp.int32, cur.shape, 1)
    cols = []
    for _ in range(kk):
        m = jnp.max(cur, axis=1, keepdims=True)
        am = jnp.min(jnp.where(cur == m, iota, n), axis=1)
        cols.append(am)
        cur = jnp.where(iota == am[:, None], -jnp.inf, cur)
    oi_ref[0] = jnp.stack(cols, axis=1)


def _topk_idx(negd, kk, tile):
    B, M, N = negd.shape
    return pl.pallas_call(
        functools.partial(_topk_kern, kk=kk),
        grid=(B, M // tile),
        in_specs=[pl.BlockSpec((1, tile, N), lambda b, i: (b, i, 0))],
        out_specs=pl.BlockSpec((1, tile, kk), lambda b, i: (b, i, 0)),
        out_shape=jax.ShapeDtypeStruct((B, M, kk), jnp.int32),
    )(negd)


def _dist_topk_kern(q_ref, k_ref, oi_ref, *, kk):
    q = q_ref[0]
    keys = k_ref[0]
    n = keys.shape[0]
    inner = lax.dot_general(
        q, keys, (((1,), (1,)), ((), ())), preferred_element_type=jnp.float32
    )
    cur = 2.0 * inner - jnp.sum(q * q, 1, keepdims=True) - jnp.sum(keys * keys, 1)[None, :]
    iota = lax.broadcasted_iota(jnp.int32, cur.shape, 1)
    cols = []
    for _ in range(kk):
        m = jnp.max(cur, axis=1, keepdims=True)
        am = jnp.min(jnp.where(cur == m, iota, n), axis=1)
        cols.append(am)
        cur = jnp.where(iota == am[:, None], -jnp.inf, cur)
    oi_ref[0] = jnp.stack(cols, axis=1)


def _dist_topk(q, k, kk, tile):
    B, M, C = q.shape
    N = k.shape[1]
    return pl.pallas_call(
        functools.partial(_dist_topk_kern, kk=kk),
        grid=(B, M // tile),
        in_specs=[
            pl.BlockSpec((1, tile, C), lambda b, i: (b, i, 0)),
            pl.BlockSpec((1, N, C), lambda b, i: (b, 0, 0)),
        ],
        out_specs=pl.BlockSpec((1, tile, kk), lambda b, i: (b, i, 0)),
        out_shape=jax.ShapeDtypeStruct((B, M, kk), jnp.int32),
    )(q, k)


def _ec_kern(g_ref, c_ref, w1_ref, w2_ref, o_ref, *, k, mode):
    acc = None
    for j in range(k):
        gj = g_ref[0, j]
        if mode == "raw":
            h = gj
        else:
            c = c_ref[0]
            e = jnp.concatenate([gj - c, c], axis=1)
            h = _act(jnp.dot(e, w1_ref[...], preferred_element_type=jnp.float32))
            if mode == "conv2":
                h = _act(jnp.dot(h, w2_ref[...], preferred_element_type=jnp.float32))
        acc = h if acc is None else jnp.maximum(acc, h)
    o_ref[0] = acc


def _edgeconv(g, center, w1_t, w2_t, mode, tile):
    B, k, N, C = g.shape
    if center is None:
        center = jnp.zeros((B, N, C), jnp.float32)
    if w1_t is None:
        w1_t = jnp.zeros((2 * C, 64), jnp.float32)
    if w2_t is None:
        w2_t = jnp.zeros((64, 64), jnp.float32)
    Co = C if mode == "raw" else w1_t.shape[1]
    return pl.pallas_call(
        functools.partial(_ec_kern, k=k, mode=mode),
        grid=(B, N // tile),
        in_specs=[
            pl.BlockSpec((1, k, tile, C), lambda b, i: (b, 0, i, 0)),
            pl.BlockSpec((1, tile, C), lambda b, i: (b, i, 0)),
            pl.BlockSpec(w1_t.shape, lambda b, i: (0, 0)),
            pl.BlockSpec(w2_t.shape, lambda b, i: (0, 0)),
        ],
        out_specs=pl.BlockSpec((1, tile, Co), lambda b, i: (b, i, 0)),
        out_shape=jax.ShapeDtypeStruct((B, N, Co), jnp.float32),
    )(g, center, w1_t, w2_t)


def _mm_kern(x_ref, w_ref, o_ref, *, act):
    acc = jnp.dot(x_ref[0], w_ref[...], preferred_element_type=jnp.float32)
    o_ref[0] = _act(acc) if act else acc


def _mm(x, w_t, act, tile):
    B, N, Ci = x.shape
    Co = w_t.shape[1]
    return pl.pallas_call(
        functools.partial(_mm_kern, act=act),
        grid=(B, N // tile),
        in_specs=[
            pl.BlockSpec((1, tile, Ci), lambda b, i: (b, i, 0)),
            pl.BlockSpec((Ci, Co), lambda b, i: (0, 0)),
        ],
        out_specs=pl.BlockSpec((1, tile, Co), lambda b, i: (b, i, 0)),
        out_shape=jax.ShapeDtypeStruct((B, N, Co), jnp.float32),
    )(x, w_t)


def _mm2_kern(x_ref, y_ref, wa_ref, wb_ref, o_ref, *, act):
    acc = jnp.dot(x_ref[0], wa_ref[...], preferred_element_type=jnp.float32)
    acc = acc + jnp.dot(y_ref[0], wb_ref[...], preferred_element_type=jnp.float32)
    o_ref[0] = _act(acc) if act else acc


def _mm2(x, y, wa_t, wb_t, act, tile):
    B, N, Ca = x.shape
    Cb = y.shape[2]
    Co = wa_t.shape[1]
    return pl.pallas_call(
        functools.partial(_mm2_kern, act=act),
        grid=(B, N // tile),
        in_specs=[
            pl.BlockSpec((1, tile, Ca), lambda b, i: (b, i, 0)),
            pl.BlockSpec((1, tile, Cb), lambda b, i: (b, i, 0)),
            pl.BlockSpec((Ca, Co), lambda b, i: (0, 0)),
            pl.BlockSpec((Cb, Co), lambda b, i: (0, 0)),
        ],
        out_specs=pl.BlockSpec((1, tile, Co), lambda b, i: (b, i, 0)),
        out_shape=jax.ShapeDtypeStruct((B, N, Co), jnp.float32),
    )(x, y, wa_t, wb_t)


def _top3_kern(q_ref, n_ref, oi_ref, ow_ref):
    q = q_ref[0]
    nodes = n_ref[0]
    M = nodes.shape[0]
    inner = lax.dot_general(
        q, nodes, (((1,), (1,)), ((), ())), preferred_element_type=jnp.float32
    )
    neg = 2.0 * inner - jnp.sum(q * q, 1, keepdims=True) - jnp.sum(nodes * nodes, 1)[None, :]
    iota = lax.broadcasted_iota(jnp.int32, neg.shape, 1)
    vals = []
    cur = neg
    for t in range(3):
        m = jnp.max(cur, axis=1, keepdims=True)
        amax = jnp.min(jnp.where(cur == m, iota, M), axis=1)
        oi_ref[0, t] = amax
        vals.append(m[:, 0])
        cur = jnp.where(iota == amax[:, None], -jnp.inf, cur)
    w = [1.0 / (jnp.maximum(-v, 0.0) + 1e-8) for v in vals]
    tot = w[0] + w[1] + w[2]
    for t in range(3):
        ow_ref[0, t] = w[t] / tot


def _unpool_top3(q, nodes, tile):
    B, N, C = q.shape
    M = nodes.shape[1]
    return pl.pallas_call(
        _top3_kern,
        grid=(B, N // tile),
        in_specs=[
            pl.BlockSpec((1, tile, C), lambda b, i: (b, i, 0)),
            pl.BlockSpec((1, M, C), lambda b, i: (b, 0, 0)),
        ],
        out_specs=[
            pl.BlockSpec((1, 3, tile), lambda b, i: (b, 0, i)),
            pl.BlockSpec((1, 3, tile), lambda b, i: (b, 0, i)),
        ],
        out_shape=[
            jax.ShapeDtypeStruct((B, 3, N), jnp.int32),
            jax.ShapeDtypeStruct((B, 3, N), jnp.float32),
        ],
    )(q, nodes)


def _final_kern(g_ref, wu_ref, x2_ref, x1_ref, wa_ref, wb_ref, wc_ref, wd_ref, o_ref):
    wu = wu_ref[0]
    hat = (
        wu[:, 0:1] * g_ref[0, 0]
        + wu[:, 1:2] * g_ref[0, 1]
        + wu[:, 2:3] * g_ref[0, 2]
    )
    h10 = jnp.dot(hat, wa_ref[...], preferred_element_type=jnp.float32)
    h10 = h10 + jnp.dot(x2_ref[0], wb_ref[...], preferred_element_type=jnp.float32)
    h10 = _act(h10)
    out = jnp.dot(h10, wc_ref[...], preferred_element_type=jnp.float32)
    out = out + jnp.dot(x1_ref[0], wd_ref[...], preferred_element_type=jnp.float32)
    o_ref[0] = out


def _final(g, wu, x2, x1, wa_t, wb_t, wc_t, wd_t, tile):
    B, _, N, D = g.shape
    Co = wc_t.shape[1]
    return pl.pallas_call(
        _final_kern,
        grid=(B, N // tile),
        in_specs=[
            pl.BlockSpec((1, 3, tile, D), lambda b, i: (b, 0, i, 0)),
            pl.BlockSpec((1, tile, 3), lambda b, i: (b, i, 0)),
            pl.BlockSpec((1, tile, 64), lambda b, i: (b, i, 0)),
            pl.BlockSpec((1, tile, 64), lambda b, i: (b, i, 0)),
            pl.BlockSpec((D, 128), lambda b, i: (0, 0)),
            pl.BlockSpec((64, 128), lambda b, i: (0, 0)),
            pl.BlockSpec((128, Co), lambda b, i: (0, 0)),
            pl.BlockSpec((64, Co), lambda b, i: (0, 0)),
        ],
        out_specs=pl.BlockSpec((1, tile, Co), lambda b, i: (b, i, 0)),
        out_shape=jax.ShapeDtypeStruct((B, N, Co), jnp.float32),
    )(g, wu, x2, x1, wa_t, wb_t, wc_t, wd_t)


def _sc_gather(table, idx):
    R, D = table.shape
    E = idx.shape[0]
    Dp = -(-D // 128) * 128
    if Dp != D:
        table = jnp.pad(table, ((0, 0), (0, Dp - D)))
    info = plsc.get_sparse_core_info()
    nc, ns = info.num_cores, info.num_subcores
    nw = nc * ns
    b_per_w = E // nw
    chunk = min(b_per_w, (98304 // Dp) // 8 * 8)
    while b_per_w % chunk:
        chunk -= 8
    nchunks = b_per_w // chunk
    mesh = plsc.VectorSubcoreMesh(core_axis_name="c", subcore_axis_name="s")

    @functools.partial(
        pl.kernel,
        mesh=mesh,
        out_type=jax.ShapeDtypeStruct((E, Dp), jnp.float32),
        scratch_types=[
            pltpu.VMEM((chunk,), jnp.int32),
            pltpu.VMEM((chunk, Dp), jnp.float32),
            pltpu.SemaphoreType.DMA,
        ],
    )
    def gk(table_hbm, idx_hbm, out_hbm, idx_v, rows_v, sem):
        wid = lax.axis_index("s") * nc + lax.axis_index("c")
        base = wid * b_per_w
        for cidx in range(nchunks):
            off = base + cidx * chunk
            pltpu.sync_copy(idx_hbm.at[pl.ds(off, chunk)], idx_v)
            pltpu.async_copy(table_hbm.at[idx_v], rows_v, sem).wait()
            pltpu.sync_copy(rows_v, out_hbm.at[pl.ds(off, chunk)])

    out = gk(table, idx)
    return out[:, :D] if Dp != D else out


def _gather_rows(table, idx, B, k, N):
    D = table.shape[1]
    flat = idx.reshape(-1)
    out = _sc_gather(table, flat)
    return out.reshape(B, k, N, D)


def kernel(x, W1, W2, W3, W4, W5, W6, W8, W9, W10, W11, p):
    B, _, N = x.shape
    k1 = _K1

    idx1 = _topk_idx(_knn_neg(x), k1, tile=256)
    h = _blk2(W2, _blk2(W1, _gfeat(x, idx1)))
    x1 = jnp.max(h, axis=-1)

    idx2 = _topk_idx(_knn_neg(x1), k1, tile=256)
    h = _blk2(W4, _blk2(W3, _gfeat(x1, idx2)))
    x2 = jnp.max(h, axis=-1)

    xyz = x[:, :3, :]
    scores = jnp.einsum("c,bcn->bn", p, x2) / (jnp.linalg.norm(p) + 1e-8)
    values, idxp = lax.top_k(scores, _NPOOL)
    feat = jnp.take_along_axis(x2, idxp[:, None, :], axis=2)
    node_feature = feat * jnp.tanh(values)[:, None, :]
    n2 = jnp.take_along_axis(xyz, idxp[:, None, :], axis=2)

    x_t = x.transpose(0, 2, 1)
    xyz_p = jnp.pad(x_t[:, :, :3], ((0, 0), (0, 0), (0, 5)))
    n2_t = n2.transpose(0, 2, 1)
    n2_p = jnp.pad(n2_t, ((0, 0), (0, 0), (0, 5)))
    x1t = x1.transpose(0, 2, 1)
    x2t = x2.transpose(0, 2, 1)
    boff1 = (jnp.arange(B, dtype=jnp.int32) * N)[:, None, None]
    boff3 = (jnp.arange(B, dtype=jnp.int32) * _NPOOL)[:, None, None]

    idxa = _dist_topk(n2_p, xyz_p, k1 // 2, tile=256)
    ga = _gather_rows(
        x2t.reshape(B * N, 64), idxa.transpose(0, 2, 1) + boff1, B, k1 // 2, _NPOOL
    )
    agg = _edgeconv(ga, None, None, None, mode="raw", tile=256)

    nf2 = jnp.concatenate([node_feature.transpose(0, 2, 1), agg], axis=2)

    idx3 = _dist_topk(nf2, nf2, k1 // 2, tile=256)
    g3 = _gather_rows(
        nf2.reshape(B * _NPOOL, 128), idx3.transpose(0, 2, 1) + boff3, B, k1 // 2, _NPOOL
    )
    x3 = _edgeconv(g3, nf2, W5.T, None, mode="act", tile=256)

    x4 = _mm(x3, W6.T, act=True, tile=256)
    h8 = _mm(x4, W8.T, act=True, tile=256)
    h9 = _mm2(h8, x3, W9[:, :256].T, W9[:, 256:].T, act=True, tile=256)

    idxu, wu = _unpool_top3(xyz_p, n2_p, tile=512)
    gu = _gather_rows(h9.reshape(B * _NPOOL, 256), idxu + boff3, B, 3, N)
    out_t = _final(
        gu, wu.transpose(0, 2, 1), x2t, x1t,
        W10[:, :256].T, W10[:, 256:].T, W11[:, :128].T, W11[:, 128:].T,
        tile=512,
    )
    out = out_t.transpose(0, 2, 1)

    return (out, scores, n2, n2)
